# baseline (speedup 1.0000x reference)
import functools

import jax
import jax.numpy as jnp
from jax.experimental import pallas as pl
from jax.experimental.pallas import tpu as pltpu

_TOPK = 2
_NID = 2
_NMLP = 6
_D = 768
_TB = 512


def _moe_body(si_ref, sw_ref, x_ref, w1_ref, b1_ref, w2_ref, b2_ref, o_ref):
    i = pl.program_id(0)
    x = x_ref[...]
    mu = jnp.mean(x, axis=1, keepdims=True)
    xc = x - mu
    var = jnp.mean(xc * xc, axis=1, keepdims=True)
    z = xc * jax.lax.rsqrt(var + 1e-5)

    si = si_ref[pl.ds(i * _TB, _TB), :]
    sw = sw_ref[pl.ds(i * _TB, _TB), :]
    acc = jnp.sum(sw, axis=1, keepdims=True) * x
    zb = z.astype(jnp.bfloat16)
    for m in range(_NMLP):
        wm = jnp.sum(jnp.where(si == (m + _NID), sw, 0.0), axis=1, keepdims=True)
        h = jnp.dot(zb, w1_ref[m], preferred_element_type=jnp.float32) + b1_ref[m]
        g = 0.5 * h * (1.0 + jax.lax.erf(h * 0.7071067811865476))
        out = jnp.dot(g.astype(jnp.bfloat16), w2_ref[m],
                      preferred_element_type=jnp.float32) + b2_ref[m]
        acc = acc + wm * out
    o_ref[...] = acc


def kernel(x, selected_indices, selected_weights, ln_g, ln_b, W1, b1, W2, b2):
    B, N, D = x.shape
    T = B * N
    xf = x.reshape(T, D)
    si = selected_indices.reshape(T, _TOPK).astype(jnp.int32)
    sw = selected_weights.reshape(T, _TOPK)
    W1f = (ln_g[:, :, None] * W1).astype(jnp.bfloat16)
    b1f = (jnp.einsum("md,mdh->mh", ln_b, W1) + b1)[:, None, :]
    b2f = b2[:, None, :]
    W2b = W2.astype(jnp.bfloat16)

    grid = (T // _TB,)
    out = pl.pallas_call(
        _moe_body,
        grid=grid,
        in_specs=[
            pl.BlockSpec((T, _TOPK), lambda i: (0, 0)),
            pl.BlockSpec((T, _TOPK), lambda i: (0, 0)),
            pl.BlockSpec((_TB, D), lambda i: (i, 0)),
            pl.BlockSpec((_NMLP, D, D), lambda i: (0, 0, 0)),
            pl.BlockSpec((_NMLP, 1, D), lambda i: (0, 0, 0)),
            pl.BlockSpec((_NMLP, D, D), lambda i: (0, 0, 0)),
            pl.BlockSpec((_NMLP, 1, D), lambda i: (0, 0, 0)),
        ],
        out_specs=pl.BlockSpec((_TB, D), lambda i: (i, 0)),
        out_shape=jax.ShapeDtypeStruct((T, D), jnp.float32),
        compiler_params=pltpu.CompilerParams(
            dimension_semantics=("arbitrary",),
        ),
    )(si, sw, xf, W1f, b1f, W2b, b2f)
    return out.reshape(B, N, D)


# speedup vs baseline: 1.0956x; 1.0956x over previous
import jax
import jax.numpy as jnp
from jax.experimental import pallas as pl
from jax.experimental.pallas import tpu as pltpu

_TOPK = 2
_NID = 2
_NMLP = 6
_TB = 512


def _moe_body(si_ref, sw_ref, x_ref, g_ref, b_ref, w1_ref, b1_ref, w2_ref,
              b2_ref, o_ref):
    x = x_ref[...]
    mu = jnp.mean(x, axis=1, keepdims=True)
    xc = x - mu
    var = jnp.mean(xc * xc, axis=1, keepdims=True)
    z = xc * jax.lax.rsqrt(var + 1e-5)

    si = si_ref[...]
    sw = sw_ref[...]
    acc = jnp.sum(sw, axis=1, keepdims=True) * x
    for m in range(_NMLP):
        wm = jnp.sum(jnp.where(si == (m + _NID), sw, 0.0), axis=1, keepdims=True)
        zm = z * g_ref[m] + b_ref[m]
        h = jnp.dot(zm, w1_ref[m], preferred_element_type=jnp.float32) + b1_ref[m]
        g = 0.5 * h * (1.0 + jax.lax.erf(h * 0.7071067811865476))
        out = jnp.dot(g, w2_ref[m], preferred_element_type=jnp.float32) + b2_ref[m]
        acc = acc + wm * out
    o_ref[...] = acc


def kernel(x, selected_indices, selected_weights, ln_g, ln_b, W1, b1, W2, b2):
    B, N, D = x.shape
    T = B * N
    xf = x.reshape(T, D)
    si = selected_indices.reshape(T, _TOPK)
    sw = selected_weights.reshape(T, _TOPK)
    gg = ln_g[:, None, :]
    bb = ln_b[:, None, :]
    b1r = b1[:, None, :]
    b2r = b2[:, None, :]

    grid = (T // _TB,)
    out = pl.pallas_call(
        _moe_body,
        grid=grid,
        in_specs=[
            pl.BlockSpec((_TB, _TOPK), lambda i: (i, 0)),
            pl.BlockSpec((_TB, _TOPK), lambda i: (i, 0)),
            pl.BlockSpec((_TB, D), lambda i: (i, 0)),
            pl.BlockSpec((_NMLP, 1, D), lambda i: (0, 0, 0)),
            pl.BlockSpec((_NMLP, 1, D), lambda i: (0, 0, 0)),
            pl.BlockSpec((_NMLP, D, D), lambda i: (0, 0, 0)),
            pl.BlockSpec((_NMLP, 1, D), lambda i: (0, 0, 0)),
            pl.BlockSpec((_NMLP, D, D), lambda i: (0, 0, 0)),
            pl.BlockSpec((_NMLP, 1, D), lambda i: (0, 0, 0)),
        ],
        out_specs=pl.BlockSpec((_TB, D), lambda i: (i, 0)),
        out_shape=jax.ShapeDtypeStruct((T, D), jnp.float32),
        compiler_params=pltpu.CompilerParams(
            dimension_semantics=("arbitrary",),
        ),
    )(si, sw, xf, gg, bb, W1, b1r, W2, b2r)
    return out.reshape(B, N, D)
